# Initial kernel scaffold; baseline (speedup 1.0000x reference)
import jax
import jax.numpy as jnp
from jax.experimental import pallas as pl


def kernel(x, b1_w1, b1_b1, b1_w2, b1_b2, b1_w3, b1_b3, b2_w1, b2_b1, b2_w2, b2_b2, b2_w3, b2_b3, b3_w1, b3_b1, b3_w2, b3_b2, b3_w3, b3_b3, b4_w1, b4_b1, b4_w2, b4_b2, b4_w3, b4_b3):
    raise NotImplementedError("write your pallas kernel here")


# speedup vs baseline: 11.5571x; 11.5571x over previous
import functools

import jax
import jax.numpy as jnp
from jax.experimental import pallas as pl
from jax.experimental.pallas import tpu as pltpu


def _maxpool_3x3_s2(h, hv, wp):
    c = h.shape[-1]
    po = (hv - 3) // 2 + 1
    g = h.reshape(hv, wp, c)
    a = g[: 2 * po].reshape(po, 2, wp, c).max(axis=1)
    b = g[2 : 2 * po + 2].reshape(po, 2, wp, c)[:, 0]
    rows = jnp.maximum(a, b)
    c1 = rows[:, : 2 * po].reshape(po, po, 2, c).max(axis=2)
    c2 = rows[:, 2 : 2 * po + 2].reshape(po, po, 2, c)[:, :, 0]
    return jnp.maximum(c1, c2)


def _block_body(x_ref, w1_ref, b1_ref, w2_ref, b2_ref, w3_ref, b3_ref, o_ref,
                *, hp, wp, hv, kh, kw, pool):
    cin = x_ref.shape[-1]
    cout = w1_ref.shape[-1]
    x = x_ref[0].reshape(hp * wp, cin)
    l = hv * wp
    acc = jnp.zeros((l, cout), jnp.float32)
    for i in range(kh):
        for j in range(kw):
            d = i * wp + j
            acc += jnp.dot(x[d : d + l], w1_ref[i * kw + j],
                           preferred_element_type=jnp.float32)
    h = jnp.maximum(acc + b1_ref[...], 0.0).astype(jnp.bfloat16)
    h = jnp.dot(h, w2_ref[...], preferred_element_type=jnp.float32)
    h = jnp.maximum(h + b2_ref[...], 0.0).astype(jnp.bfloat16)
    h = jnp.dot(h, w3_ref[...], preferred_element_type=jnp.float32)
    h = jnp.maximum(h + b3_ref[...], 0.0).astype(jnp.bfloat16)
    if pool == "max":
        o_ref[0] = _maxpool_3x3_s2(h, hv, wp)
    else:
        g = h.reshape(hv, wp, cout)[:, :hv].astype(jnp.float32)
        o_ref[0] = g.sum(axis=0).sum(axis=0, keepdims=True) / (hv * hv)


def _run_block(x, w1, b1, w2, b2, w3, b3, *, hp, wp, hv, kh, kw, pool):
    n, cin = x.shape[0], x.shape[-1]
    cout = w1.shape[-1]
    body = functools.partial(_block_body, hp=hp, wp=wp, hv=hv, kh=kh, kw=kw,
                             pool=pool)
    if pool == "max":
        po = (hv - 3) // 2 + 1
        out_shape = jax.ShapeDtypeStruct((n, po, po, cout), jnp.bfloat16)
        out_spec = pl.BlockSpec((1, po, po, cout), lambda i: (i, 0, 0, 0))
    else:
        out_shape = jax.ShapeDtypeStruct((n, 1, cout), jnp.float32)
        out_spec = pl.BlockSpec((1, 1, cout), lambda i: (i, 0, 0))
    return pl.pallas_call(
        body,
        out_shape=out_shape,
        grid=(n,),
        in_specs=[
            pl.BlockSpec((1, hp, wp, cin), lambda i: (i, 0, 0, 0)),
            pl.BlockSpec(w1.shape, lambda i: (0, 0, 0)),
            pl.BlockSpec(b1.shape, lambda i: (0, 0)),
            pl.BlockSpec(w2.shape, lambda i: (0, 0)),
            pl.BlockSpec(b2.shape, lambda i: (0, 0)),
            pl.BlockSpec(w3.shape, lambda i: (0, 0)),
            pl.BlockSpec(b3.shape, lambda i: (0, 0)),
        ],
        out_specs=out_spec,
        compiler_params=pltpu.CompilerParams(
            dimension_semantics=("parallel",)),
    )(x, w1, b1, w2, b2, w3, b3)


def _space_to_depth4(x_nchw):
    n = x_nchw.shape[0]
    x = jnp.transpose(x_nchw, (0, 2, 3, 1))
    x = x.reshape(n, 56, 4, 56, 4, 3).transpose(0, 1, 3, 2, 4, 5)
    x = x.reshape(n, 56, 56, 48)
    x = jnp.pad(x, ((0, 0), (0, 1), (0, 0), (0, 0)))
    return x.astype(jnp.bfloat16)


def _prep_w1_b1(w):
    w = w[:363].reshape(11, 11, 3, 128)
    w = jnp.pad(w, ((0, 1), (0, 1), (0, 0), (0, 0)))
    w = w.reshape(3, 4, 3, 4, 3, 128).transpose(0, 2, 1, 3, 4, 5)
    return w.reshape(9, 48, 128)


def kernel(x, b1_w1, b1_b1, b1_w2, b1_b2, b1_w3, b1_b3,
           b2_w1, b2_b1, b2_w2, b2_b2, b2_w3, b2_b3,
           b3_w1, b3_b1, b3_w2, b3_b2, b3_w3, b3_b3,
           b4_w1, b4_b1, b4_w2, b4_b2, b4_w3, b4_b3):
    w1b1 = _prep_w1_b1(b1_w1)
    w1b2 = jnp.pad(b2_w1[:2400].reshape(25, 96, 256), ((0, 0), (0, 32), (0, 0)))
    w1b3 = b3_w1[:2304].reshape(9, 256, 384)
    w1b4 = b4_w1[:3456].reshape(9, 384, 128)

    x1 = _space_to_depth4(x)
    p1 = _run_block(x1, w1b1, b1_b1, b1_w2, b1_b2, b1_w3, b1_b3,
                    hp=57, wp=56, hv=54, kh=3, kw=3, pool="max")
    x2 = jnp.pad(p1, ((0, 0), (2, 3), (2, 4), (0, 0)))
    p2 = _run_block(x2, w1b2, b2_b1, b2_w2, b2_b2, b2_w3, b2_b3,
                    hp=31, wp=32, hv=26, kh=5, kw=5, pool="max")
    x3 = jnp.pad(p2, ((0, 0), (1, 2), (1, 3), (0, 0)))
    p3 = _run_block(x3, w1b3, b3_b1, b3_w2, b3_b2, b3_w3, b3_b3,
                    hp=15, wp=16, hv=12, kh=3, kw=3, pool="max")
    x4 = jnp.pad(p3, ((0, 0), (1, 2), (1, 2), (0, 0)))
    out = _run_block(x4, w1b4, b4_b1, b4_w2, b4_b2, b4_w3, b4_b3,
                     hp=8, wp=8, hv=5, kh=3, kw=3, pool="avg")
    return out[:, 0, :10]


# speedup vs baseline: 11.8315x; 1.0237x over previous
import functools

import jax
import jax.numpy as jnp
from jax.experimental import pallas as pl
from jax.experimental.pallas import tpu as pltpu


def _maxpool_3x3_s2(h, hv, wp):
    c = h.shape[-1]
    po = (hv - 3) // 2 + 1
    g = h.reshape(hv, wp, c)
    a = g[: 2 * po].reshape(po, 2, wp, c).max(axis=1)
    b = g[2 : 2 * po + 2].reshape(po, 2, wp, c)[:, 0]
    rows = jnp.maximum(a, b)
    c1 = rows[:, : 2 * po].reshape(po, po, 2, c).max(axis=2)
    c2 = rows[:, 2 : 2 * po + 2].reshape(po, po, 2, c)[:, :, 0]
    return jnp.maximum(c1, c2)


def _block_body(x_ref, w1_ref, b1_ref, w2_ref, b2_ref, w3_ref, b3_ref, o_ref,
                *, hp, wp, hv, kh, kw, pool, opad):
    cin = x_ref.shape[-1]
    cout = w1_ref.shape[-1]
    x = x_ref[0].reshape(hp * wp, cin)
    l = hv * wp
    acc = jnp.zeros((l, cout), jnp.float32)
    for i in range(kh):
        for j in range(kw):
            d = i * wp + j
            acc += jnp.dot(x[d : d + l], w1_ref[i * kw + j],
                           preferred_element_type=jnp.float32)
    h = jnp.maximum(acc + b1_ref[...], 0.0).astype(jnp.bfloat16)
    h = jnp.dot(h, w2_ref[...], preferred_element_type=jnp.float32)
    h = jnp.maximum(h + b2_ref[...], 0.0).astype(jnp.bfloat16)
    h = jnp.dot(h, w3_ref[...], preferred_element_type=jnp.float32)
    h = jnp.maximum(h + b3_ref[...], 0.0).astype(jnp.bfloat16)
    if pool == "max":
        po = (hv - 3) // 2 + 1
        o_ref[0] = jnp.zeros(o_ref.shape[1:], o_ref.dtype)
        o_ref[0, opad : opad + po, opad : opad + po, :] = _maxpool_3x3_s2(
            h, hv, wp)
    else:
        g = h.reshape(hv, wp, cout)[:, :hv].astype(jnp.float32)
        o_ref[0] = g.sum(axis=0).sum(axis=0, keepdims=True) / (hv * hv)


def _run_block(x, w1, b1, w2, b2, w3, b3, *, hp, wp, hv, kh, kw, pool,
               oh=0, ow=0, opad=0):
    n, cin = x.shape[0], x.shape[-1]
    cout = w1.shape[-1]
    body = functools.partial(_block_body, hp=hp, wp=wp, hv=hv, kh=kh, kw=kw,
                             pool=pool, opad=opad)
    if pool == "max":
        out_shape = jax.ShapeDtypeStruct((n, oh, ow, cout), jnp.bfloat16)
        out_spec = pl.BlockSpec((1, oh, ow, cout), lambda i: (i, 0, 0, 0))
    else:
        out_shape = jax.ShapeDtypeStruct((n, 1, cout), jnp.float32)
        out_spec = pl.BlockSpec((1, 1, cout), lambda i: (i, 0, 0))
    return pl.pallas_call(
        body,
        out_shape=out_shape,
        grid=(n,),
        in_specs=[
            pl.BlockSpec((1, hp, wp, cin), lambda i: (i, 0, 0, 0)),
            pl.BlockSpec(w1.shape, lambda i: (0, 0, 0)),
            pl.BlockSpec(b1.shape, lambda i: (0, 0)),
            pl.BlockSpec(w2.shape, lambda i: (0, 0)),
            pl.BlockSpec(b2.shape, lambda i: (0, 0)),
            pl.BlockSpec(w3.shape, lambda i: (0, 0)),
            pl.BlockSpec(b3.shape, lambda i: (0, 0)),
        ],
        out_specs=out_spec,
        compiler_params=pltpu.CompilerParams(
            dimension_semantics=("parallel",)),
    )(x, w1, b1, w2, b2, w3, b3)


def _space_to_depth4(x_nchw):
    n = x_nchw.shape[0]
    x = jnp.transpose(x_nchw, (0, 2, 3, 1))
    x = x.reshape(n, 56, 4, 56, 4, 3).transpose(0, 1, 3, 2, 4, 5)
    x = x.reshape(n, 56, 56, 48)
    x = jnp.pad(x, ((0, 0), (0, 1), (0, 0), (0, 0)))
    return x.astype(jnp.bfloat16)


def _prep_w1_b1(w):
    w = w[:363].reshape(11, 11, 3, 128)
    w = jnp.pad(w, ((0, 1), (0, 1), (0, 0), (0, 0)))
    w = w.reshape(3, 4, 3, 4, 3, 128).transpose(0, 2, 1, 3, 4, 5)
    return w.reshape(9, 48, 128)


def kernel(x, b1_w1, b1_b1, b1_w2, b1_b2, b1_w3, b1_b3,
           b2_w1, b2_b1, b2_w2, b2_b2, b2_w3, b2_b3,
           b3_w1, b3_b1, b3_w2, b3_b2, b3_w3, b3_b3,
           b4_w1, b4_b1, b4_w2, b4_b2, b4_w3, b4_b3):
    w1b1 = _prep_w1_b1(b1_w1)
    w1b2 = jnp.pad(b2_w1[:2400].reshape(25, 96, 256), ((0, 0), (0, 32), (0, 0)))
    w1b3 = b3_w1[:2304].reshape(9, 256, 384)
    w1b4 = b4_w1[:3456].reshape(9, 384, 128)

    x1 = _space_to_depth4(x)
    x2 = _run_block(x1, w1b1, b1_b1, b1_w2, b1_b2, b1_w3, b1_b3,
                    hp=57, wp=56, hv=54, kh=3, kw=3, pool="max",
                    oh=31, ow=32, opad=2)
    x3 = _run_block(x2, w1b2, b2_b1, b2_w2, b2_b2, b2_w3, b2_b3,
                    hp=31, wp=32, hv=26, kh=5, kw=5, pool="max",
                    oh=15, ow=16, opad=1)
    x4 = _run_block(x3, w1b3, b3_b1, b3_w2, b3_b2, b3_w3, b3_b3,
                    hp=15, wp=16, hv=12, kh=3, kw=3, pool="max",
                    oh=8, ow=8, opad=1)
    out = _run_block(x4, w1b4, b4_b1, b4_w2, b4_b2, b4_w3, b4_b3,
                     hp=8, wp=8, hv=5, kh=3, kw=3, pool="avg")
    return out[:, 0, :10]
